# baseline (speedup 1.0000x reference)
import functools

import jax
import jax.numpy as jnp
from jax import lax
from jax.experimental import pallas as pl
from jax.experimental.pallas import tpu as pltpu
from jax.experimental.pallas import tpu_sc as plsc

TAU0 = 0.5
NC = 2
NS = 16
LANES = 128


def _copy_edge_slice(ei_hbm, base, full, tail, buf, padrow_hbm, sem):
  descs = []
  for j in range(full):
    descs.append(
        pltpu.async_copy(ei_hbm.at[pl.ds(base + j * LANES, LANES)],
                         buf.at[j], sem))
  if tail:
    descs.append(
        pltpu.async_copy(ei_hbm.at[pl.ds(base + full * LANES, tail)],
                         buf.at[full, pl.ds(0, tail)], sem))
    descs.append(
        pltpu.async_copy(padrow_hbm.at[pl.ds(0, LANES - tail)],
                         buf.at[full, pl.ds(tail, LANES - tail)], sem))
  return descs


def _make_deg_kernel(npad, e):
  mesh = plsc.VectorSubcoreMesh(
      core_axis_name="c", subcore_axis_name="s", num_cores=NC,
      num_subcores=NS)
  sl = npad // NS
  ept = e // (NC * NS)
  full = ept // LANES
  tail = ept - full * LANES
  erows = full + (1 if tail else 0)

  @functools.partial(
      pl.kernel,
      out_type=jax.ShapeDtypeStruct((NC, npad), jnp.float32),
      mesh=mesh,
      scratch_types=[
          pltpu.VMEM((erows, LANES), jnp.int32),
          pltpu.VMEM((LANES,), jnp.float32),
          pltpu.VMEM((sl,), jnp.float32),
          pltpu.VMEM_SHARED((npad,), jnp.float32),
          pltpu.SemaphoreType.DMA,
          pltpu.SemaphoreType.DMA,
      ],
  )
  def deg_kernel(ei_hbm, zeros_hbm, ones_hbm, padrow_hbm, out_hbm,
                 dst_v, ones_v, stage_v, hist_sh, sem_in, sem):
    c = lax.axis_index("c")
    s = lax.axis_index("s")
    w = c * NS + s

    indescs = _copy_edge_slice(ei_hbm, w * ept, full, tail, dst_v,
                               padrow_hbm, sem_in)
    pltpu.sync_copy(ones_hbm, ones_v)

    @pl.when(s == 0)
    def _():
      pltpu.sync_copy(zeros_hbm, hist_sh)

    for d in indescs:
      d.wait()
    plsc.subcore_barrier()

    descs = []
    for j in range(erows):
      descs.append(
          pltpu.async_copy(ones_v, hist_sh.at[dst_v.at[j]], sem, add=True))
    for d in descs:
      d.wait()

    plsc.subcore_barrier()
    base = s * sl
    pltpu.sync_copy(hist_sh.at[pl.ds(base, sl)], stage_v)
    pltpu.sync_copy(stage_v, out_hbm.at[c, pl.ds(base, sl)])

  return deg_kernel


def _make_edge_kernel(npad, e):
  mesh = plsc.VectorSubcoreMesh(
      core_axis_name="c", subcore_axis_name="s", num_cores=NC,
      num_subcores=NS)
  sl = npad // NS
  srows = sl // LANES
  ept = e // (NC * NS)
  full = ept // LANES
  tail = ept - full * LANES
  erows = full + (1 if tail else 0)

  @functools.partial(
      pl.kernel,
      out_type=jax.ShapeDtypeStruct((NC, npad), jnp.float32),
      mesh=mesh,
      scratch_types=[
          pltpu.VMEM((erows, LANES), jnp.int32),
          pltpu.VMEM((erows, LANES), jnp.int32),
          pltpu.VMEM((erows, LANES), jnp.float32),
          pltpu.VMEM((sl,), jnp.float32),
          pltpu.VMEM((sl,), jnp.float32),
          pltpu.VMEM((sl,), jnp.float32),
          pltpu.VMEM((sl,), jnp.float32),
          pltpu.VMEM((srows, LANES), jnp.int32),
          pltpu.VMEM((sl,), jnp.float32),
          pltpu.VMEM_SHARED((npad,), jnp.float32),
          pltpu.VMEM_SHARED((npad,), jnp.float32),
          pltpu.SemaphoreType.DMA,
          pltpu.SemaphoreType.DMA,
          pltpu.SemaphoreType.DMA,
      ],
  )
  def edge_kernel(src_hbm, dst_hbm, hist_hbm, h_hbm, tab_hbm, zeros_hbm,
                  padrow_hbm, out_hbm,
                  src_v, dst_v, vals_v, hv, d0, d1, gv, idx_v, dinv_v,
                  g_sh, acc_sh, sem_in, sem_g, sem_s):
    c = lax.axis_index("c")
    s = lax.axis_index("s")
    w = c * NS + s
    base = s * sl

    indescs = _copy_edge_slice(src_hbm, w * ept, full, tail, src_v,
                               padrow_hbm, sem_in)
    indescs += _copy_edge_slice(dst_hbm, w * ept, full, tail, dst_v,
                                padrow_hbm, sem_in)

    @pl.when(s == 0)
    def _():
      pltpu.sync_copy(zeros_hbm, acc_sh)

    pltpu.sync_copy(h_hbm.at[pl.ds(base, sl)], hv)
    pltpu.sync_copy(hist_hbm.at[0, pl.ds(base, sl)], d0)
    pltpu.sync_copy(hist_hbm.at[1, pl.ds(base, sl)], d1)
    for k in range(sl // 16):
      ix = pl.ds(k * 16, 16)
      deg = d0[ix] + d1[ix] + 1.0
      idx_v[k // 8, pl.ds((k % 8) * 16, 16)] = deg.astype(jnp.int32)
    tdescs = []
    for r in range(srows):
      tdescs.append(
          pltpu.async_copy(tab_hbm.at[idx_v.at[r]],
                           dinv_v.at[pl.ds(r * LANES, LANES)], sem_g))
    for d in tdescs:
      d.wait()
    for k in range(sl // 16):
      ix = pl.ds(k * 16, 16)
      gv[ix] = dinv_v[ix] * hv[ix]
    pltpu.sync_copy(gv, g_sh.at[pl.ds(base, sl)])

    for d in indescs:
      d.wait()
    plsc.subcore_barrier()

    gdescs = []
    for j in range(erows):
      gdescs.append(
          pltpu.async_copy(g_sh.at[src_v.at[j]], vals_v.at[j], sem_g))
    for d in gdescs:
      d.wait()
    sdescs = []
    for j in range(erows):
      sdescs.append(
          pltpu.async_copy(vals_v.at[j], acc_sh.at[dst_v.at[j]], sem_s,
                           add=True))
    for d in sdescs:
      d.wait()

    plsc.subcore_barrier()
    pltpu.sync_copy(acc_sh.at[pl.ds(base, sl)], gv)
    pltpu.sync_copy(gv, out_hbm.at[c, pl.ds(base, sl)])

  return edge_kernel


def _matvec(x, w_row):
  n, d = x.shape
  blocks = 5
  rb = n // blocks

  def body(x_ref, w_ref, o_ref):
    i = pl.program_id(0)
    s = lax.dot_general(w_ref[...], x_ref[...], (((1,), (1,)), ((), ())),
                        preferred_element_type=jnp.float32)
    o_ref[pl.ds(i, 1), :] = s

  return pl.pallas_call(
      body,
      grid=(blocks,),
      in_specs=[
          pl.BlockSpec((rb, d), lambda i: (i, 0)),
          pl.BlockSpec((1, d), lambda i: (0, 0)),
      ],
      out_specs=pl.BlockSpec((blocks, rb), lambda i: (0, 0)),
      out_shape=jax.ShapeDtypeStruct((blocks, rb), jnp.float32),
  )(x, w_row)


def _epilogue(hist, acc, h_row):
  npad = h_row.shape[1]

  def body(hist_ref, acc_ref, h_ref, o_ref):
    deg = hist_ref[0:1, :] + hist_ref[1:2, :] + 1.0
    dinv = lax.rsqrt(deg)
    a = acc_ref[0:1, :] + acc_ref[1:2, :]
    out = dinv * a + dinv * dinv * h_ref[...]
    sp = jnp.maximum(out, 0.0) + jnp.log1p(jnp.exp(-jnp.abs(out))) + TAU0
    o_ref[...] = 1.0 / sp

  return pl.pallas_call(
      body,
      out_shape=jax.ShapeDtypeStruct((1, npad), jnp.float32),
  )(hist, acc, h_row)


def kernel(x, edge_index, edge_attr, W):
  n = x.shape[0]
  e = edge_index.shape[1]
  npad = (n // 256 + 1) * 256
  npads = npad - n

  zeros = jnp.zeros((npad,), jnp.float32)
  ones = jnp.ones((LANES,), jnp.float32)
  padrow = n + (jnp.arange(LANES, dtype=jnp.int32) % npads)
  tab = lax.rsqrt(jnp.arange(e + 2, dtype=jnp.float32))

  src1 = edge_index[0]
  dst1 = edge_index[1]
  hist = _make_deg_kernel(npad, e)(dst1, zeros, ones, padrow)

  h = _matvec(x, W.reshape(1, -1))
  h_lin = jnp.concatenate([h.reshape(n), jnp.zeros((npads,), jnp.float32)])

  acc = _make_edge_kernel(npad, e)(src1, dst1, hist, h_lin, tab, zeros,
                                   padrow)

  temp = _epilogue(hist, acc, h_lin.reshape(1, npad))
  return temp[0, :n, None]


# speedup vs baseline: 3.0153x; 3.0153x over previous
import functools

import jax
import jax.numpy as jnp
from jax import lax
from jax.experimental import pallas as pl
from jax.experimental.pallas import tpu as pltpu
from jax.experimental.pallas import tpu_sc as plsc

TAU0 = 0.5
NC = 2
NS = 16
LANES = 128


def _copy_edge_slice(ei_hbm, base, full, tail, buf, padrow_hbm, sem):
  descs = []
  for j in range(full):
    descs.append(
        pltpu.async_copy(ei_hbm.at[pl.ds(base + j * LANES, LANES)],
                         buf.at[j], sem))
  if tail:
    descs.append(
        pltpu.async_copy(ei_hbm.at[pl.ds(base + full * LANES, tail)],
                         buf.at[full, pl.ds(0, tail)], sem))
    descs.append(
        pltpu.async_copy(padrow_hbm.at[pl.ds(0, LANES - tail)],
                         buf.at[full, pl.ds(tail, LANES - tail)], sem))
  return descs


def _make_deg_kernel(npad, e):
  mesh = plsc.VectorSubcoreMesh(
      core_axis_name="c", subcore_axis_name="s", num_cores=NC,
      num_subcores=NS)
  sl = npad // NS
  ept = e // (NC * NS)
  full = ept // LANES
  tail = ept - full * LANES
  erows = full + (1 if tail else 0)

  @functools.partial(
      pl.kernel,
      out_type=jax.ShapeDtypeStruct((NC, npad), jnp.float32),
      mesh=mesh,
      scratch_types=[
          pltpu.VMEM((erows, LANES), jnp.int32),
          pltpu.VMEM((LANES,), jnp.float32),
          pltpu.VMEM((sl,), jnp.float32),
          pltpu.VMEM_SHARED((npad,), jnp.float32),
          pltpu.SemaphoreType.DMA,
          pltpu.SemaphoreType.DMA,
      ],
  )
  def deg_kernel(ei_hbm, zeros_hbm, ones_hbm, padrow_hbm, out_hbm,
                 dst_v, ones_v, stage_v, hist_sh, sem_in, sem):
    c = lax.axis_index("c")
    s = lax.axis_index("s")
    w = c * NS + s

    indescs = _copy_edge_slice(ei_hbm, e + w * ept, full, tail, dst_v,
                               padrow_hbm, sem_in)
    pltpu.sync_copy(ones_hbm, ones_v)

    @pl.when(s == 0)
    def _():
      pltpu.sync_copy(zeros_hbm, hist_sh)

    for d in indescs:
      d.wait()
    plsc.subcore_barrier()

    descs = []
    for j in range(erows):
      descs.append(
          pltpu.async_copy(ones_v, hist_sh.at[dst_v.at[j]], sem, add=True))
    for d in descs:
      d.wait()

    plsc.subcore_barrier()
    base = s * sl
    pltpu.sync_copy(hist_sh.at[pl.ds(base, sl)], stage_v)
    pltpu.sync_copy(stage_v, out_hbm.at[c, pl.ds(base, sl)])

  return deg_kernel


def _make_edge_kernel(npad, e):
  mesh = plsc.VectorSubcoreMesh(
      core_axis_name="c", subcore_axis_name="s", num_cores=NC,
      num_subcores=NS)
  sl = npad // NS
  ept = e // (NC * NS)
  full = ept // LANES
  tail = ept - full * LANES
  erows = full + (1 if tail else 0)

  @functools.partial(
      pl.kernel,
      out_type=jax.ShapeDtypeStruct((NC, npad), jnp.float32),
      mesh=mesh,
      scratch_types=[
          pltpu.VMEM((erows, LANES), jnp.int32),
          pltpu.VMEM((erows, LANES), jnp.int32),
          pltpu.VMEM((erows, LANES), jnp.float32),
          pltpu.VMEM((sl,), jnp.float32),
          pltpu.VMEM_SHARED((npad,), jnp.float32),
          pltpu.VMEM_SHARED((npad,), jnp.float32),
          pltpu.SemaphoreType.DMA,
          pltpu.SemaphoreType.DMA,
          pltpu.SemaphoreType.DMA,
      ],
  )
  def edge_kernel(ei_hbm, g_hbm, zeros_hbm, padrow_hbm, out_hbm,
                  src_v, dst_v, vals_v, gv,
                  g_sh, acc_sh, sem_in, sem_g, sem_s):
    c = lax.axis_index("c")
    s = lax.axis_index("s")
    w = c * NS + s
    base = s * sl

    indescs = _copy_edge_slice(ei_hbm, w * ept, full, tail, src_v,
                               padrow_hbm, sem_in)
    indescs += _copy_edge_slice(ei_hbm, e + w * ept, full, tail, dst_v,
                                padrow_hbm, sem_in)

    @pl.when(s == 0)
    def _():
      pltpu.sync_copy(zeros_hbm, acc_sh)

    pltpu.sync_copy(g_hbm.at[0, pl.ds(base, sl)], gv)
    pltpu.sync_copy(gv, g_sh.at[pl.ds(base, sl)])

    for d in indescs:
      d.wait()
    plsc.subcore_barrier()

    gdescs = []
    for j in range(erows):
      gdescs.append(
          pltpu.async_copy(g_sh.at[src_v.at[j]], vals_v.at[j], sem_g))
    for d in gdescs:
      d.wait()
    sdescs = []
    for j in range(erows):
      sdescs.append(
          pltpu.async_copy(vals_v.at[j], acc_sh.at[dst_v.at[j]], sem_s,
                           add=True))
    for d in sdescs:
      d.wait()

    plsc.subcore_barrier()
    pltpu.sync_copy(acc_sh.at[pl.ds(base, sl)], gv)
    pltpu.sync_copy(gv, out_hbm.at[c, pl.ds(base, sl)])

  return edge_kernel


def _matvec(x, w_row):
  n, d = x.shape
  blocks = 5
  rb = n // blocks

  def body(x_ref, w_ref, o_ref):
    i = pl.program_id(0)
    s = lax.dot_general(w_ref[...], x_ref[...], (((1,), (1,)), ((), ())),
                        preferred_element_type=jnp.float32)
    o_ref[pl.ds(i, 1), :] = s

  return pl.pallas_call(
      body,
      grid=(blocks,),
      in_specs=[
          pl.BlockSpec((rb, d), lambda i: (i, 0)),
          pl.BlockSpec((1, d), lambda i: (0, 0)),
      ],
      out_specs=pl.BlockSpec((blocks, rb), lambda i: (0, 0)),
      out_shape=jax.ShapeDtypeStruct((blocks, rb), jnp.float32),
  )(x, w_row)


def _gcompute(hist, h_row):
  npad = h_row.shape[1]

  def body(hist_ref, h_ref, g_ref):
    deg = hist_ref[0:1, :] + hist_ref[1:2, :] + 1.0
    g_ref[...] = lax.rsqrt(deg) * h_ref[...]

  return pl.pallas_call(
      body,
      out_shape=jax.ShapeDtypeStruct((1, npad), jnp.float32),
  )(hist, h_row)


def _epilogue(hist, acc, h_row):
  npad = h_row.shape[1]

  def body(hist_ref, acc_ref, h_ref, o_ref):
    deg = hist_ref[0:1, :] + hist_ref[1:2, :] + 1.0
    dinv = lax.rsqrt(deg)
    a = acc_ref[0:1, :] + acc_ref[1:2, :]
    out = dinv * a + dinv * dinv * h_ref[...]
    sp = jnp.maximum(out, 0.0) + jnp.log1p(jnp.exp(-jnp.abs(out))) + TAU0
    o_ref[...] = 1.0 / sp

  return pl.pallas_call(
      body,
      out_shape=jax.ShapeDtypeStruct((1, npad), jnp.float32),
  )(hist, acc, h_row)


def kernel(x, edge_index, edge_attr, W):
  n = x.shape[0]
  e = edge_index.shape[1]
  npad = (n // 256 + 1) * 256
  npads = npad - n

  zeros = jnp.zeros((npad,), jnp.float32)
  ones = jnp.ones((LANES,), jnp.float32)
  padrow = n + (jnp.arange(LANES, dtype=jnp.int32) % npads)

  flat = edge_index.reshape(2 * e)
  hist = _make_deg_kernel(npad, e)(flat, zeros, ones, padrow)

  h = _matvec(x, W.reshape(1, -1))
  h_row = jnp.concatenate([h.reshape(1, n),
                           jnp.zeros((1, npads), jnp.float32)], axis=1)
  g_row = _gcompute(hist, h_row)

  acc = _make_edge_kernel(npad, e)(flat, g_row, zeros, padrow)

  temp = _epilogue(hist, acc, h_row)
  return temp[0, :n, None]


# speedup vs baseline: 3.7390x; 1.2400x over previous
import functools

import jax
import jax.numpy as jnp
from jax import lax
from jax.experimental import pallas as pl
from jax.experimental.pallas import tpu as pltpu
from jax.experimental.pallas import tpu_sc as plsc

TAU0 = 0.5
NC = 2
NS = 16
NW = NC * NS
LANES = 128
CH = 8


def _zero_fill(buf, nwords):
  z = jnp.zeros((16,), jnp.float32)
  for k in range(nwords // 16):
    buf[pl.ds(k * 16, 16)] = z


def _make_deg_kernel(npad, e):
  mesh = plsc.VectorSubcoreMesh(
      core_axis_name="c", subcore_axis_name="s", num_cores=NC,
      num_subcores=NS)
  sl = npad // NS
  fr = e // LANES
  q, r = fr // NW, fr % NW

  @functools.partial(
      pl.kernel,
      out_type=jax.ShapeDtypeStruct((NC, npad), jnp.float32),
      mesh=mesh,
      scratch_types=[
          pltpu.VMEM((q + 1, LANES), jnp.int32),
          pltpu.VMEM((LANES,), jnp.float32),
          pltpu.VMEM((sl,), jnp.float32),
          pltpu.VMEM_SHARED((npad,), jnp.float32),
          pltpu.SemaphoreType.DMA,
          pltpu.SemaphoreType.DMA,
      ],
  )
  def deg_kernel(ei_hbm, out_hbm, dst_v, ones_v, stage_v, hist_sh,
                 sem_in, sem):
    c = lax.axis_index("c")
    s = lax.axis_index("s")
    w = c * NS + s
    extra = w < r
    ebase = e + (w * q + jnp.minimum(w, r)) * LANES

    indescs = []
    for j in range(q):
      indescs.append(
          pltpu.async_copy(ei_hbm.at[pl.ds(ebase + j * LANES, LANES)],
                           dst_v.at[j], sem_in))

    @pl.when(extra)
    def _():
      pltpu.async_copy(ei_hbm.at[pl.ds(ebase + q * LANES, LANES)],
                       dst_v.at[q], sem_in)

    one = jnp.ones((16,), jnp.float32)
    for k in range(LANES // 16):
      ones_v[pl.ds(k * 16, 16)] = one

    nbase = s * sl
    _zero_fill(stage_v, sl)
    pltpu.sync_copy(stage_v, hist_sh.at[pl.ds(nbase, sl)])
    plsc.subcore_barrier()

    descs = []
    for j in range(q):
      indescs[j].wait()
      descs.append(
          pltpu.async_copy(ones_v, hist_sh.at[dst_v.at[j]], sem, add=True))

    @pl.when(extra)
    def _():
      pltpu.make_async_copy(ei_hbm.at[pl.ds(ebase + q * LANES, LANES)],
                            dst_v.at[q], sem_in).wait()
      pltpu.async_copy(ones_v, hist_sh.at[dst_v.at[q]], sem, add=True).wait()

    for d in descs:
      d.wait()

    plsc.subcore_barrier()
    pltpu.sync_copy(hist_sh.at[pl.ds(nbase, sl)], stage_v)
    pltpu.sync_copy(stage_v, out_hbm.at[c, pl.ds(nbase, sl)])

  return deg_kernel


def _make_edge_kernel(npad, e):
  mesh = plsc.VectorSubcoreMesh(
      core_axis_name="c", subcore_axis_name="s", num_cores=NC,
      num_subcores=NS)
  sl = npad // NS
  fr = e // LANES
  q, r = fr // NW, fr % NW

  @functools.partial(
      pl.kernel,
      out_type=jax.ShapeDtypeStruct((NC, npad), jnp.float32),
      mesh=mesh,
      scratch_types=[
          pltpu.VMEM((q + 1, LANES), jnp.int32),
          pltpu.VMEM((q + 1, LANES), jnp.int32),
          pltpu.VMEM((q + 1, LANES), jnp.float32),
          pltpu.VMEM((sl,), jnp.float32),
          pltpu.VMEM((sl,), jnp.float32),
          pltpu.VMEM_SHARED((npad,), jnp.float32),
          pltpu.VMEM_SHARED((npad,), jnp.float32),
          pltpu.SemaphoreType.DMA,
          pltpu.SemaphoreType.DMA,
          pltpu.SemaphoreType.DMA,
      ],
  )
  def edge_kernel(ei_hbm, g_hbm, out_hbm,
                  src_v, dst_v, vals_v, gv, zv,
                  g_sh, acc_sh, sem_in, sem_g, sem_s):
    c = lax.axis_index("c")
    s = lax.axis_index("s")
    w = c * NS + s
    extra = w < r
    base0 = (w * q + jnp.minimum(w, r)) * LANES

    indescs = []
    for j in range(q):
      indescs.append(
          pltpu.async_copy(ei_hbm.at[pl.ds(base0 + j * LANES, LANES)],
                           src_v.at[j], sem_in))
      indescs.append(
          pltpu.async_copy(ei_hbm.at[pl.ds(e + base0 + j * LANES, LANES)],
                           dst_v.at[j], sem_in))

    @pl.when(extra)
    def _():
      pltpu.async_copy(ei_hbm.at[pl.ds(base0 + q * LANES, LANES)],
                       src_v.at[q], sem_in)
      pltpu.async_copy(ei_hbm.at[pl.ds(e + base0 + q * LANES, LANES)],
                       dst_v.at[q], sem_in)

    nbase = s * sl
    _zero_fill(zv, sl)
    pltpu.sync_copy(zv, acc_sh.at[pl.ds(nbase, sl)])
    pltpu.sync_copy(g_hbm.at[0, pl.ds(nbase, sl)], gv)
    pltpu.sync_copy(gv, g_sh.at[pl.ds(nbase, sl)])
    plsc.subcore_barrier()

    chunks = [range(i, min(i + CH, q)) for i in range(0, q, CH)]
    gdescs = [None] * q
    sdescs = []
    for ci in range(len(chunks) + 1):
      if ci < len(chunks):
        for j in chunks[ci]:
          indescs[2 * j].wait()
          indescs[2 * j + 1].wait()
          gdescs[j] = pltpu.async_copy(g_sh.at[src_v.at[j]], vals_v.at[j],
                                       sem_g)
      if ci >= 1:
        for j in chunks[ci - 1]:
          gdescs[j].wait()
          sdescs.append(
              pltpu.async_copy(vals_v.at[j], acc_sh.at[dst_v.at[j]], sem_s,
                               add=True))

    @pl.when(extra)
    def _():
      pltpu.make_async_copy(ei_hbm.at[pl.ds(base0 + q * LANES, LANES)],
                            src_v.at[q], sem_in).wait()
      pltpu.make_async_copy(ei_hbm.at[pl.ds(e + base0 + q * LANES, LANES)],
                            dst_v.at[q], sem_in).wait()
      pltpu.async_copy(g_sh.at[src_v.at[q]], vals_v.at[q], sem_g).wait()
      pltpu.async_copy(vals_v.at[q], acc_sh.at[dst_v.at[q]], sem_s,
                       add=True).wait()

    for d in sdescs:
      d.wait()

    plsc.subcore_barrier()
    pltpu.sync_copy(acc_sh.at[pl.ds(nbase, sl)], gv)
    pltpu.sync_copy(gv, out_hbm.at[c, pl.ds(nbase, sl)])

  return edge_kernel


def _matvec(x, w_row):
  n, d = x.shape
  blocks = 5
  rb = n // blocks

  def body(x_ref, w_ref, o_ref):
    i = pl.program_id(0)
    s = lax.dot_general(w_ref[...], x_ref[...], (((1,), (1,)), ((), ())),
                        preferred_element_type=jnp.float32)
    o_ref[pl.ds(i, 1), :] = s

  return pl.pallas_call(
      body,
      grid=(blocks,),
      in_specs=[
          pl.BlockSpec((rb, d), lambda i: (i, 0)),
          pl.BlockSpec((1, d), lambda i: (0, 0)),
      ],
      out_specs=pl.BlockSpec((blocks, rb), lambda i: (0, 0)),
      out_shape=jax.ShapeDtypeStruct((blocks, rb), jnp.float32),
  )(x, w_row)


def _gcompute(hist, h_row):
  npad = h_row.shape[1]

  def body(hist_ref, h_ref, g_ref):
    deg = hist_ref[0:1, :] + hist_ref[1:2, :] + 1.0
    g_ref[...] = lax.rsqrt(deg) * h_ref[...]

  return pl.pallas_call(
      body,
      out_shape=jax.ShapeDtypeStruct((1, npad), jnp.float32),
  )(hist, h_row)


def _epilogue(hist, acc, h_row):
  npad = h_row.shape[1]

  def body(hist_ref, acc_ref, h_ref, o_ref):
    deg = hist_ref[0:1, :] + hist_ref[1:2, :] + 1.0
    dinv = lax.rsqrt(deg)
    a = acc_ref[0:1, :] + acc_ref[1:2, :]
    out = dinv * a + dinv * dinv * h_ref[...]
    sp = jnp.maximum(out, 0.0) + jnp.log1p(jnp.exp(-jnp.abs(out))) + TAU0
    o_ref[...] = 1.0 / sp

  return pl.pallas_call(
      body,
      out_shape=jax.ShapeDtypeStruct((1, npad), jnp.float32),
  )(hist, acc, h_row)


def kernel(x, edge_index, edge_attr, W):
  n = x.shape[0]
  e = edge_index.shape[1]
  npad = (n // 256 + 1) * 256
  npads = npad - n

  flat = edge_index.reshape(2 * e)
  hist = _make_deg_kernel(npad, e)(flat)

  h = _matvec(x, W.reshape(1, -1))
  h_row = jnp.concatenate([h.reshape(1, n),
                           jnp.zeros((1, npads), jnp.float32)], axis=1)
  g_row = _gcompute(hist, h_row)

  acc = _make_edge_kernel(npad, e)(flat, g_row)

  temp = _epilogue(hist, acc, h_row)
  return temp[0, :n, None]


# speedup vs baseline: 3.8492x; 1.0295x over previous
import functools

import jax
import jax.numpy as jnp
from jax import lax
from jax.experimental import pallas as pl
from jax.experimental.pallas import tpu as pltpu
from jax.experimental.pallas import tpu_sc as plsc

TAU0 = 0.5
NC = 2
NS = 16
NW = NC * NS
LANES = 128
CH = 8


def _zero_fill(buf, nwords):
  z = jnp.zeros((16,), jnp.float32)
  for k in range(nwords // 16):
    buf[pl.ds(k * 16, 16)] = z


def _make_deg_kernel(npad, e):
  mesh = plsc.VectorSubcoreMesh(
      core_axis_name="c", subcore_axis_name="s", num_cores=NC,
      num_subcores=NS)
  sl = npad // NS
  fr = e // LANES
  q, r = fr // NW, fr % NW

  @functools.partial(
      pl.kernel,
      out_type=jax.ShapeDtypeStruct((NC, npad), jnp.float32),
      mesh=mesh,
      scratch_types=[
          pltpu.VMEM((q + 1, 2, LANES), jnp.int32),
          pltpu.VMEM((LANES,), jnp.float32),
          pltpu.VMEM((sl,), jnp.float32),
          pltpu.VMEM_SHARED((npad,), jnp.float32),
          pltpu.SemaphoreType.DMA,
          pltpu.SemaphoreType.DMA,
      ],
  )
  def deg_kernel(ei_hbm, out_hbm, ei_v, ones_v, stage_v, hist_sh,
                 sem_in, sem):
    c = lax.axis_index("c")
    s = lax.axis_index("s")
    w = c * NS + s
    extra = w < r
    col0 = (w * q + jnp.minimum(w, r)) * LANES

    indescs = []
    for j in range(q):
      indescs.append(
          pltpu.async_copy(ei_hbm.at[:, pl.ds(col0 + j * LANES, LANES)],
                           ei_v.at[j], sem_in))

    @pl.when(extra)
    def _():
      pltpu.async_copy(ei_hbm.at[:, pl.ds(col0 + q * LANES, LANES)],
                       ei_v.at[q], sem_in)

    one = jnp.ones((16,), jnp.float32)
    for k in range(LANES // 16):
      ones_v[pl.ds(k * 16, 16)] = one

    nbase = s * sl
    _zero_fill(stage_v, sl)
    pltpu.sync_copy(stage_v, hist_sh.at[pl.ds(nbase, sl)])
    plsc.subcore_barrier()

    descs = []
    for j in range(q):
      indescs[j].wait()
      descs.append(
          pltpu.async_copy(ones_v, hist_sh.at[ei_v.at[j, 1]], sem, add=True))

    @pl.when(extra)
    def _():
      pltpu.make_async_copy(ei_hbm.at[:, pl.ds(col0 + q * LANES, LANES)],
                            ei_v.at[q], sem_in).wait()
      pltpu.async_copy(ones_v, hist_sh.at[ei_v.at[q, 1]], sem,
                       add=True).wait()

    for d in descs:
      d.wait()

    plsc.subcore_barrier()
    pltpu.sync_copy(hist_sh.at[pl.ds(nbase, sl)], stage_v)
    pltpu.sync_copy(stage_v, out_hbm.at[c, pl.ds(nbase, sl)])

  return deg_kernel


def _make_edge_kernel(npad, e):
  mesh = plsc.VectorSubcoreMesh(
      core_axis_name="c", subcore_axis_name="s", num_cores=NC,
      num_subcores=NS)
  sl = npad // NS
  fr = e // LANES
  q, r = fr // NW, fr % NW

  @functools.partial(
      pl.kernel,
      out_type=jax.ShapeDtypeStruct((NC, npad), jnp.float32),
      mesh=mesh,
      scratch_types=[
          pltpu.VMEM((q + 1, 2, LANES), jnp.int32),
          pltpu.VMEM((q + 1, LANES), jnp.float32),
          pltpu.VMEM((sl,), jnp.float32),
          pltpu.VMEM((sl,), jnp.float32),
          pltpu.VMEM_SHARED((npad,), jnp.float32),
          pltpu.VMEM_SHARED((npad,), jnp.float32),
          pltpu.SemaphoreType.DMA,
          pltpu.SemaphoreType.DMA,
          pltpu.SemaphoreType.DMA,
      ],
  )
  def edge_kernel(ei_hbm, g_hbm, out_hbm,
                  ei_v, vals_v, gv, zv,
                  g_sh, acc_sh, sem_in, sem_g, sem_s):
    c = lax.axis_index("c")
    s = lax.axis_index("s")
    w = c * NS + s
    extra = w < r
    col0 = (w * q + jnp.minimum(w, r)) * LANES

    indescs = []
    for j in range(q):
      indescs.append(
          pltpu.async_copy(ei_hbm.at[:, pl.ds(col0 + j * LANES, LANES)],
                           ei_v.at[j], sem_in))

    @pl.when(extra)
    def _():
      pltpu.async_copy(ei_hbm.at[:, pl.ds(col0 + q * LANES, LANES)],
                       ei_v.at[q], sem_in)

    nbase = s * sl
    _zero_fill(zv, sl)
    pltpu.sync_copy(zv, acc_sh.at[pl.ds(nbase, sl)])
    pltpu.sync_copy(g_hbm.at[0, pl.ds(nbase, sl)], gv)
    pltpu.sync_copy(gv, g_sh.at[pl.ds(nbase, sl)])
    plsc.subcore_barrier()

    chunks = [range(i, min(i + CH, q)) for i in range(0, q, CH)]
    gdescs = [None] * q
    sdescs = []
    for ci in range(len(chunks) + 1):
      if ci < len(chunks):
        for j in chunks[ci]:
          indescs[j].wait()
          gdescs[j] = pltpu.async_copy(g_sh.at[ei_v.at[j, 0]], vals_v.at[j],
                                       sem_g)
      if ci >= 1:
        for j in chunks[ci - 1]:
          gdescs[j].wait()
          sdescs.append(
              pltpu.async_copy(vals_v.at[j], acc_sh.at[ei_v.at[j, 1]], sem_s,
                               add=True))

    @pl.when(extra)
    def _():
      pltpu.make_async_copy(ei_hbm.at[:, pl.ds(col0 + q * LANES, LANES)],
                            ei_v.at[q], sem_in).wait()
      pltpu.async_copy(g_sh.at[ei_v.at[q, 0]], vals_v.at[q], sem_g).wait()
      pltpu.async_copy(vals_v.at[q], acc_sh.at[ei_v.at[q, 1]], sem_s,
                       add=True).wait()

    for d in sdescs:
      d.wait()

    plsc.subcore_barrier()
    pltpu.sync_copy(acc_sh.at[pl.ds(nbase, sl)], gv)
    pltpu.sync_copy(gv, out_hbm.at[c, pl.ds(nbase, sl)])

  return edge_kernel


def _matvec(x, w_row):
  n, d = x.shape
  blocks = 10
  rb = n // blocks

  def body(x_ref, w_ref, o_ref):
    i = pl.program_id(0)
    s = lax.dot_general(w_ref[...], x_ref[...], (((1,), (1,)), ((), ())),
                        preferred_element_type=jnp.float32)
    o_ref[pl.ds(i, 1), :] = s

  return pl.pallas_call(
      body,
      grid=(blocks,),
      in_specs=[
          pl.BlockSpec((rb, d), lambda i: (i, 0)),
          pl.BlockSpec((1, d), lambda i: (0, 0)),
      ],
      out_specs=pl.BlockSpec((blocks, rb), lambda i: (0, 0)),
      out_shape=jax.ShapeDtypeStruct((blocks, rb), jnp.float32),
  )(x, w_row)


def _gcompute(hist, h_row):
  npad = h_row.shape[1]

  def body(hist_ref, h_ref, g_ref):
    deg = hist_ref[0:1, :] + hist_ref[1:2, :] + 1.0
    g_ref[...] = lax.rsqrt(deg) * h_ref[...]

  return pl.pallas_call(
      body,
      out_shape=jax.ShapeDtypeStruct((1, npad), jnp.float32),
  )(hist, h_row)


def _epilogue(hist, acc, h_row):
  npad = h_row.shape[1]

  def body(hist_ref, acc_ref, h_ref, o_ref):
    deg = hist_ref[0:1, :] + hist_ref[1:2, :] + 1.0
    dinv = lax.rsqrt(deg)
    a = acc_ref[0:1, :] + acc_ref[1:2, :]
    out = dinv * a + dinv * dinv * h_ref[...]
    sp = jnp.maximum(out, 0.0) + jnp.log1p(jnp.exp(-jnp.abs(out))) + TAU0
    o_ref[...] = 1.0 / sp

  return pl.pallas_call(
      body,
      out_shape=jax.ShapeDtypeStruct((1, npad), jnp.float32),
  )(hist, acc, h_row)


def kernel(x, edge_index, edge_attr, W):
  n = x.shape[0]
  e = edge_index.shape[1]
  npad = (n // 256 + 1) * 256
  npads = npad - n

  hist = _make_deg_kernel(npad, e)(edge_index)

  h = _matvec(x, W.reshape(1, -1))
  h_row = jnp.concatenate([h.reshape(1, n),
                           jnp.zeros((1, npads), jnp.float32)], axis=1)
  g_row = _gcompute(hist, h_row)

  acc = _make_edge_kernel(npad, e)(edge_index, g_row)

  temp = _epilogue(hist, acc, h_row)
  return temp[0, :n, None]


# speedup vs baseline: 4.0472x; 1.0514x over previous
import functools

import jax
import jax.numpy as jnp
from jax import lax
from jax.experimental import pallas as pl
from jax.experimental.pallas import tpu as pltpu
from jax.experimental.pallas import tpu_sc as plsc

TAU0 = 0.5
NC = 2
NS = 16
NW = NC * NS
LANES = 128
CH = 8


def _zero_fill(buf, nwords):
  z = jnp.zeros((16,), jnp.float32)
  for k in range(nwords // 16):
    buf[pl.ds(k * 16, 16)] = z


def _make_deg_kernel(npad, e):
  mesh = plsc.VectorSubcoreMesh(
      core_axis_name="c", subcore_axis_name="s", num_cores=NC,
      num_subcores=NS)
  sl = npad // NS
  fr = e // LANES
  q, r = fr // NW, fr % NW

  @functools.partial(
      pl.kernel,
      out_type=jax.ShapeDtypeStruct((NC, npad), jnp.float32),
      mesh=mesh,
      scratch_types=[
          pltpu.VMEM((q + 1, 2, LANES), jnp.int32),
          pltpu.VMEM((LANES,), jnp.float32),
          pltpu.VMEM((sl,), jnp.float32),
          pltpu.VMEM_SHARED((npad,), jnp.float32),
          pltpu.SemaphoreType.DMA,
          pltpu.SemaphoreType.DMA,
      ],
  )
  def deg_kernel(ei_hbm, out_hbm, ei_v, ones_v, stage_v, hist_sh,
                 sem_in, sem):
    c = lax.axis_index("c")
    s = lax.axis_index("s")
    w = c * NS + s
    extra = w < r
    col0 = (w * q + jnp.minimum(w, r)) * LANES

    indescs = []
    for j in range(q):
      indescs.append(
          pltpu.async_copy(ei_hbm.at[:, pl.ds(col0 + j * LANES, LANES)],
                           ei_v.at[j], sem_in))

    @pl.when(extra)
    def _():
      pltpu.async_copy(ei_hbm.at[:, pl.ds(col0 + q * LANES, LANES)],
                       ei_v.at[q], sem_in)

    one = jnp.ones((16,), jnp.float32)
    for k in range(LANES // 16):
      ones_v[pl.ds(k * 16, 16)] = one

    nbase = s * sl
    _zero_fill(stage_v, sl)
    pltpu.sync_copy(stage_v, hist_sh.at[pl.ds(nbase, sl)])
    plsc.subcore_barrier()

    descs = []
    for j in range(q):
      indescs[j].wait()
      descs.append(
          pltpu.async_copy(ones_v, hist_sh.at[ei_v.at[j, 1]], sem, add=True))

    @pl.when(extra)
    def _():
      pltpu.make_async_copy(ei_hbm.at[:, pl.ds(col0 + q * LANES, LANES)],
                            ei_v.at[q], sem_in).wait()
      pltpu.async_copy(ones_v, hist_sh.at[ei_v.at[q, 1]], sem,
                       add=True).wait()

    for d in descs:
      d.wait()

    plsc.subcore_barrier()
    pltpu.sync_copy(hist_sh.at[pl.ds(nbase, sl)], stage_v)
    pltpu.sync_copy(stage_v, out_hbm.at[c, pl.ds(nbase, sl)])

  return deg_kernel


def _make_edge_kernel(npad, e):
  mesh = plsc.VectorSubcoreMesh(
      core_axis_name="c", subcore_axis_name="s", num_cores=NC,
      num_subcores=NS)
  sl = npad // NS
  fr = e // LANES
  q, r = fr // NW, fr % NW

  @functools.partial(
      pl.kernel,
      out_type=jax.ShapeDtypeStruct((NC, npad), jnp.float32),
      mesh=mesh,
      scratch_types=[
          pltpu.VMEM((q + 1, 2, LANES), jnp.int32),
          pltpu.VMEM((q + 1, LANES), jnp.float32),
          pltpu.VMEM((sl,), jnp.float32),
          pltpu.VMEM((sl,), jnp.float32),
          pltpu.VMEM_SHARED((npad,), jnp.float32),
          pltpu.VMEM_SHARED((npad,), jnp.float32),
          pltpu.SemaphoreType.DMA,
          pltpu.SemaphoreType.DMA,
          pltpu.SemaphoreType.DMA,
      ],
  )
  def edge_kernel(ei_hbm, g_hbm, out_hbm,
                  ei_v, vals_v, gv, zv,
                  g_sh, acc_sh, sem_in, sem_g, sem_s):
    c = lax.axis_index("c")
    s = lax.axis_index("s")
    w = c * NS + s
    extra = w < r
    col0 = (w * q + jnp.minimum(w, r)) * LANES

    indescs = []
    for j in range(q):
      indescs.append(
          pltpu.async_copy(ei_hbm.at[:, pl.ds(col0 + j * LANES, LANES)],
                           ei_v.at[j], sem_in))

    @pl.when(extra)
    def _():
      pltpu.async_copy(ei_hbm.at[:, pl.ds(col0 + q * LANES, LANES)],
                       ei_v.at[q], sem_in)

    nbase = s * sl
    _zero_fill(zv, sl)
    pltpu.sync_copy(zv, acc_sh.at[pl.ds(nbase, sl)])
    pltpu.sync_copy(g_hbm.at[0, pl.ds(nbase, sl)], gv)
    pltpu.sync_copy(gv, g_sh.at[pl.ds(nbase, sl)])
    plsc.subcore_barrier()

    chunks = [range(i, min(i + CH, q)) for i in range(0, q, CH)]
    gdescs = [None] * q
    sdescs = []
    for ci in range(len(chunks) + 1):
      if ci < len(chunks):
        for j in chunks[ci]:
          indescs[j].wait()
          gdescs[j] = pltpu.async_copy(g_sh.at[ei_v.at[j, 0]], vals_v.at[j],
                                       sem_g)
      if ci >= 1:
        for j in chunks[ci - 1]:
          gdescs[j].wait()
          sdescs.append(
              pltpu.async_copy(vals_v.at[j], acc_sh.at[ei_v.at[j, 1]], sem_s,
                               add=True))

    @pl.when(extra)
    def _():
      pltpu.make_async_copy(ei_hbm.at[:, pl.ds(col0 + q * LANES, LANES)],
                            ei_v.at[q], sem_in).wait()
      pltpu.async_copy(g_sh.at[ei_v.at[q, 0]], vals_v.at[q], sem_g).wait()
      pltpu.async_copy(vals_v.at[q], acc_sh.at[ei_v.at[q, 1]], sem_s,
                       add=True).wait()

    for d in sdescs:
      d.wait()

    plsc.subcore_barrier()
    pltpu.sync_copy(acc_sh.at[pl.ds(nbase, sl)], zv)
    pltpu.sync_copy(zv, out_hbm.at[c, pl.ds(nbase, sl)])

  return edge_kernel


def _matvec(x, w_row):
  n, d = x.shape
  blocks = 5
  rb = n // blocks

  def body(x_ref, w_ref, o_ref):
    i = pl.program_id(0)
    s = lax.dot_general(w_ref[...], x_ref[...], (((1,), (1,)), ((), ())),
                        preferred_element_type=jnp.float32)
    o_ref[pl.ds(i, 1), :] = s

  return pl.pallas_call(
      body,
      grid=(blocks,),
      in_specs=[
          pl.BlockSpec((rb, d), lambda i: (i, 0)),
          pl.BlockSpec((1, d), lambda i: (0, 0)),
      ],
      out_specs=pl.BlockSpec((blocks, rb), lambda i: (0, 0)),
      out_shape=jax.ShapeDtypeStruct((blocks, rb), jnp.float32),
  )(x, w_row)


def _gcompute(hist, h_row):
  npad = h_row.shape[1]

  def body(hist_ref, h_ref, g_ref):
    deg = hist_ref[0:1, :] + hist_ref[1:2, :] + 1.0
    g_ref[...] = lax.rsqrt(deg) * h_ref[...]

  return pl.pallas_call(
      body,
      out_shape=jax.ShapeDtypeStruct((1, npad), jnp.float32),
  )(hist, h_row)


def _epilogue(hist, acc, h_row):
  npad = h_row.shape[1]

  def body(hist_ref, acc_ref, h_ref, o_ref):
    deg = hist_ref[0:1, :] + hist_ref[1:2, :] + 1.0
    dinv = lax.rsqrt(deg)
    a = acc_ref[0:1, :] + acc_ref[1:2, :]
    out = dinv * a + dinv * dinv * h_ref[...]
    sp = jnp.maximum(out, 0.0) + jnp.log1p(jnp.exp(-jnp.abs(out))) + TAU0
    o_ref[...] = 1.0 / sp

  return pl.pallas_call(
      body,
      out_shape=jax.ShapeDtypeStruct((1, npad), jnp.float32),
  )(hist, acc, h_row)


def kernel(x, edge_index, edge_attr, W):
  n = x.shape[0]
  e = edge_index.shape[1]
  npad = (n // 256 + 1) * 256
  npads = npad - n

  hist = _make_deg_kernel(npad, e)(edge_index)

  h = _matvec(x, W.reshape(1, -1))
  h_row = jnp.concatenate([h.reshape(1, n),
                           jnp.zeros((1, npads), jnp.float32)], axis=1)
  g_row = _gcompute(hist, h_row)

  acc = _make_edge_kernel(npad, e)(edge_index, g_row)

  temp = _epilogue(hist, acc, h_row)
  return temp[0, :n, None]
